# baseline (speedup 1.0000x reference)
import functools

import jax
import jax.numpy as jnp
from jax import lax
from jax.experimental import pallas as pl
from jax.experimental.pallas import tpu as pltpu
from jax.experimental.pallas import tpu_sc as plsc

B = 16384
S = 50
D = 32
L = 16

_NC = 2
_NS = 16
NW = _NC * _NS
RW = B // NW
CB = 16
NCHUNK = RW // CB


def _rsqrt_nr(x):
    i = plsc.bitcast(x, jnp.int32)
    i = jnp.int32(0x5F3759DF) - (i >> 1)
    y = plsc.bitcast(i, jnp.float32)
    for _ in range(3):
        y = y * (1.5 - 0.5 * x * y * y)
    return y


@functools.partial(
    pl.kernel,
    out_type=(
        jax.ShapeDtypeStruct((B * 3,), jnp.float32),
        jax.ShapeDtypeStruct((B * D,), jnp.float32),
    ),
    mesh=plsc.VectorSubcoreMesh(core_axis_name="c", subcore_axis_name="s"),
    compiler_params=pltpu.CompilerParams(
        needs_layout_passes=False, use_tc_tiling_on_sc=False),
    scratch_types=(
        pltpu.VMEM((2, CB, S), jnp.int32),
        pltpu.VMEM((2, CB, S, D), jnp.bfloat16),
        pltpu.VMEM((RW,), jnp.int32),
        pltpu.VMEM((8 + D,), jnp.float32),
        pltpu.VMEM((D * L,), jnp.float32),
        pltpu.VMEM((CB * D,), jnp.float32),
        pltpu.VMEM((RW * 3,), jnp.float32),
        pltpu.VMEM((RW * D,), jnp.float32),
        pltpu.SemaphoreType.DMA,
        pltpu.SemaphoreType.DMA,
        pltpu.SemaphoreType.DMA,
        pltpu.SemaphoreType.DMA,
    ),
)
def _sc_encoder(value_hbm, status_hbm, table_hbm, rep_hbm,
                short_hbm, full_hbm,
                idx_v, rows_v, stat_v, rep_v, repb_v, pooled_v, outs_v, outf_v,
                gsem0, gsem1, isem0, isem1):
    wid = lax.axis_index("s") * _NC + lax.axis_index("c")
    base = wid * RW
    gsem = (gsem0, gsem1)
    isem = (isem0, isem1)

    pltpu.sync_copy(status_hbm.at[pl.ds(base, RW)], stat_v)
    pltpu.sync_copy(rep_hbm, rep_v.at[pl.ds(8, D)])
    for k in range(D):
        kv8 = jnp.full((L,), 8 + k, jnp.int32)
        repb_v[pl.ds(k * L, L)] = plsc.load_gather(rep_v, [kv8])

    lane = lax.iota(jnp.int32, L)
    lane_d = lane * D
    lane_3 = lane * 3
    inv_s_const = jnp.float32(1.0 / S)

    def start_idx(c, b):
        pltpu.async_copy(value_hbm.at[pl.ds(base + c * CB, CB)], idx_v.at[b],
                         isem[b])

    def drain_idx(b):
        pltpu.make_async_copy(value_hbm.at[pl.ds(0, CB)], idx_v.at[b],
                              isem[b]).wait()

    def fire_gathers(b):
        for r in range(CB):
            pltpu.async_copy(table_hbm.at[idx_v.at[b, r]], rows_v.at[b, r],
                             gsem[b])

    def drain_gathers(b):
        for r in range(CB):
            pltpu.make_async_copy(table_hbm.at[idx_v.at[b, r]],
                                  rows_v.at[b, r], gsem[b]).wait()

    def process(c, b):
        zero = jnp.zeros((L,), jnp.float32)
        for r in range(CB):
            def jbody(j, acc, _r=r, _b=b):
                a0, b0, a1, b1 = acc
                for u in range(5):
                    j2 = j * 10 + u * 2
                    e0, o0 = plsc.unpack(rows_v[_b, _r, j2, :],
                                         format=plsc.PackFormat.INTERLEAVED)
                    e1, o1 = plsc.unpack(rows_v[_b, _r, j2 + 1, :],
                                         format=plsc.PackFormat.INTERLEAVED)
                    a0 = a0 + e0
                    a1 = a1 + o0
                    b0 = b0 + e1
                    b1 = b1 + o1
                return (a0, b0, a1, b1)
            a0, b0, a1, b1 = lax.fori_loop(0, 5, jbody,
                                           (zero, zero, zero, zero))
            pooled_v[pl.ds(r * D, L)] = (a0 + b0) * inv_s_const
            pooled_v[pl.ds(r * D + L, L)] = (a1 + b1) * inv_s_const

        sv = stat_v[pl.ds(c * CB, CB)]
        msk = sv != 0
        ssq_f = jnp.zeros((L,), jnp.float32)
        ssq_s = jnp.zeros((L,), jnp.float32)
        for k in range(D):
            pk = (k % 2) * L + k // 2
            kv = jnp.full((L,), pk, jnp.int32)
            col = plsc.load_gather(pooled_v, [lane_d + kv])
            repk = repb_v[pl.ds(k * L, L)]
            col = jnp.where(msk, repk, col)
            ssq_f = ssq_f + col * col
            if k < 3:
                ssq_s = ssq_s + col * col
        inv_f = _rsqrt_nr(jnp.maximum(ssq_f, jnp.float32(1e-16)))
        inv_3 = _rsqrt_nr(jnp.maximum(ssq_s, jnp.float32(1e-16)))
        grow_d = (c * CB) * D + lane_d
        grow_3 = (c * CB) * 3 + lane_3
        for k in range(D):
            pk = (k % 2) * L + k // 2
            col = plsc.load_gather(pooled_v, [lane_d + jnp.full((L,), pk,
                                                                jnp.int32)])
            repk = repb_v[pl.ds(k * L, L)]
            col = jnp.where(msk, repk, col)
            kv = jnp.full((L,), k, jnp.int32)
            plsc.store_scatter(outf_v, [grow_d + kv], col * inv_f)
            if k < 3:
                plsc.store_scatter(outs_v, [grow_3 + kv], col * inv_3)

    start_idx(0, 0)
    drain_idx(0)
    fire_gathers(0)
    start_idx(1, 1)

    def body(i, carry):
        c0 = 2 * i

        drain_idx(1)
        fire_gathers(1)
        drain_gathers(0)

        @pl.when(c0 + 2 < NCHUNK)
        def _():
            start_idx(c0 + 2, 0)

        process(c0, 0)

        @pl.when(c0 + 2 < NCHUNK)
        def _():
            drain_idx(0)
            fire_gathers(0)
        drain_gathers(1)

        @pl.when(c0 + 3 < NCHUNK)
        def _():
            start_idx(c0 + 3, 1)

        process(c0 + 1, 1)
        return carry

    lax.fori_loop(0, NCHUNK // 2, body, 0)

    pltpu.sync_copy(outf_v, full_hbm.at[pl.ds(base * D, RW * D)])
    pltpu.sync_copy(outs_v, short_hbm.at[pl.ds(base * 3, RW * 3)])


def kernel(value, status, table, replacement):
    value = value.astype(jnp.int32)
    status = status.astype(jnp.int32)
    replacement = replacement.astype(jnp.float32)
    table_bf = table.astype(jnp.bfloat16)
    short_flat, full_flat = _sc_encoder(value, status, table_bf, replacement)
    return short_flat.reshape(B, 3), full_flat.reshape(B, D)


# speedup vs baseline: 1.1668x; 1.1668x over previous
import functools

import jax
import jax.numpy as jnp
from jax import lax
from jax.experimental import pallas as pl
from jax.experimental.pallas import tpu as pltpu
from jax.experimental.pallas import tpu_sc as plsc

B = 16384
S = 50
D = 32
L = 16

_NC = 2
_NS = 16
NW = _NC * _NS
RW = B // NW
CB = 16
NCHUNK = RW // CB


def _rsqrt_nr(x):
    i = plsc.bitcast(x, jnp.int32)
    i = jnp.int32(0x5F3759DF) - (i >> 1)
    y = plsc.bitcast(i, jnp.float32)
    for _ in range(3):
        y = y * (1.5 - 0.5 * x * y * y)
    return y


@functools.partial(
    pl.kernel,
    out_type=(
        jax.ShapeDtypeStruct((B * 3,), jnp.float32),
        jax.ShapeDtypeStruct((B * D,), jnp.float32),
    ),
    mesh=plsc.VectorSubcoreMesh(core_axis_name="c", subcore_axis_name="s"),
    compiler_params=pltpu.CompilerParams(
        needs_layout_passes=False, use_tc_tiling_on_sc=False),
    scratch_types=(
        pltpu.VMEM((2, CB, S), jnp.int32),
        pltpu.VMEM((2, CB, S, D), jnp.float32),
        pltpu.VMEM((RW,), jnp.int32),
        pltpu.VMEM((8 + D,), jnp.float32),
        pltpu.VMEM((D * L,), jnp.float32),
        pltpu.VMEM((CB * D,), jnp.float32),
        pltpu.VMEM((RW * 3,), jnp.float32),
        pltpu.VMEM((RW * D,), jnp.float32),
        pltpu.SemaphoreType.DMA,
        pltpu.SemaphoreType.DMA,
        pltpu.SemaphoreType.DMA,
        pltpu.SemaphoreType.DMA,
    ),
)
def _sc_encoder(value_hbm, status_hbm, table_hbm, rep_hbm,
                short_hbm, full_hbm,
                idx_v, rows_v, stat_v, rep_v, repb_v, pooled_v, outs_v, outf_v,
                gsem0, gsem1, isem0, isem1):
    wid = lax.axis_index("s") * _NC + lax.axis_index("c")
    base = wid * RW
    gsem = (gsem0, gsem1)
    isem = (isem0, isem1)

    pltpu.sync_copy(status_hbm.at[pl.ds(base, RW)], stat_v)
    pltpu.sync_copy(rep_hbm, rep_v.at[pl.ds(8, D)])
    for k in range(D):
        kv8 = jnp.full((L,), 8 + k, jnp.int32)
        repb_v[pl.ds(k * L, L)] = plsc.load_gather(rep_v, [kv8])

    lane = lax.iota(jnp.int32, L)
    lane_d = lane * D
    lane_3 = lane * 3
    inv_s_const = jnp.float32(1.0 / S)

    def start_idx(c, b):
        pltpu.async_copy(value_hbm.at[pl.ds(base + c * CB, CB)], idx_v.at[b],
                         isem[b])

    def drain_idx(b):
        pltpu.make_async_copy(value_hbm.at[pl.ds(0, CB)], idx_v.at[b],
                              isem[b]).wait()

    def fire_gathers(b):
        for r in range(CB):
            pltpu.async_copy(table_hbm.at[idx_v.at[b, r]], rows_v.at[b, r],
                             gsem[b])

    def drain_gathers(b):
        for r in range(CB):
            pltpu.make_async_copy(table_hbm.at[idx_v.at[b, r]],
                                  rows_v.at[b, r], gsem[b]).wait()

    def process(c, b):
        zero = jnp.zeros((L,), jnp.float32)
        for r in range(CB):
            def jbody(j, acc, _r=r, _b=b):
                a0, b0, a1, b1 = acc
                for u in range(5):
                    j2 = j * 10 + u * 2
                    a0 = a0 + rows_v[_b, _r, j2, pl.ds(0, L)]
                    a1 = a1 + rows_v[_b, _r, j2, pl.ds(L, L)]
                    b0 = b0 + rows_v[_b, _r, j2 + 1, pl.ds(0, L)]
                    b1 = b1 + rows_v[_b, _r, j2 + 1, pl.ds(L, L)]
                return (a0, b0, a1, b1)
            a0, b0, a1, b1 = lax.fori_loop(0, 5, jbody,
                                           (zero, zero, zero, zero))
            pooled_v[pl.ds(r * D, L)] = (a0 + b0) * inv_s_const
            pooled_v[pl.ds(r * D + L, L)] = (a1 + b1) * inv_s_const

        sv = stat_v[pl.ds(c * CB, CB)]
        msk = sv != 0
        ssq_f = jnp.zeros((L,), jnp.float32)
        ssq_s = jnp.zeros((L,), jnp.float32)
        for k in range(D):
            kv = jnp.full((L,), k, jnp.int32)
            col = plsc.load_gather(pooled_v, [lane_d + kv])
            repk = repb_v[pl.ds(k * L, L)]
            col = jnp.where(msk, repk, col)
            ssq_f = ssq_f + col * col
            if k < 3:
                ssq_s = ssq_s + col * col
        inv_f = _rsqrt_nr(jnp.maximum(ssq_f, jnp.float32(1e-16)))
        inv_3 = _rsqrt_nr(jnp.maximum(ssq_s, jnp.float32(1e-16)))
        grow_d = (c * CB) * D + lane_d
        grow_3 = (c * CB) * 3 + lane_3
        for k in range(D):
            kv = jnp.full((L,), k, jnp.int32)
            col = plsc.load_gather(pooled_v, [lane_d + kv])
            repk = repb_v[pl.ds(k * L, L)]
            col = jnp.where(msk, repk, col)
            plsc.store_scatter(outf_v, [grow_d + kv], col * inv_f)
            if k < 3:
                plsc.store_scatter(outs_v, [grow_3 + kv], col * inv_3)

    start_idx(0, 0)
    drain_idx(0)
    fire_gathers(0)
    start_idx(1, 1)

    def body(i, carry):
        c0 = 2 * i

        drain_idx(1)
        fire_gathers(1)
        drain_gathers(0)

        @pl.when(c0 + 2 < NCHUNK)
        def _():
            start_idx(c0 + 2, 0)

        process(c0, 0)

        @pl.when(c0 + 2 < NCHUNK)
        def _():
            drain_idx(0)
            fire_gathers(0)
        drain_gathers(1)

        @pl.when(c0 + 3 < NCHUNK)
        def _():
            start_idx(c0 + 3, 1)

        process(c0 + 1, 1)
        return carry

    lax.fori_loop(0, NCHUNK // 2, body, 0)

    pltpu.sync_copy(outf_v, full_hbm.at[pl.ds(base * D, RW * D)])
    pltpu.sync_copy(outs_v, short_hbm.at[pl.ds(base * 3, RW * 3)])


def kernel(value, status, table, replacement):
    value = value.astype(jnp.int32)
    status = status.astype(jnp.int32)
    table = table.astype(jnp.float32)
    replacement = replacement.astype(jnp.float32)
    short_flat, full_flat = _sc_encoder(value, status, table, replacement)
    return short_flat.reshape(B, 3), full_flat.reshape(B, D)
